# baseline (speedup 1.0000x reference)
import functools

import jax
import jax.numpy as jnp
from jax import lax
from jax.experimental import pallas as pl
from jax.experimental.pallas import tpu as pltpu
from jax.experimental.pallas import tpu_sc as plsc

N_USERS = 1_000_000
N_ITEMS = 100_000
F = 16
B_U = 16384
B_I = 4096

_NC = 2
_NS = 16
_NW = _NC * _NS

_U_PER_W = B_U // _NW
_I_PER_W = B_I // _NW
_L = 16


def _gather_rows(table_hbm, idx_ref, rows_ref, sem, n_rows):
  def group(g, _):
    vec = idx_ref[pl.ds(g * _L, _L)]
    for l in range(_L):
      r = vec[l]
      pltpu.async_copy(
          table_hbm.at[pl.ds(r, 1)],
          rows_ref.at[pl.ds(g * _L + l, 1)], sem)
    return 0

  lax.fori_loop(0, n_rows // _L, group, 0)


def _drain(table_hbm, rows_ref, sem, n_rows):
  pltpu.make_async_copy(
      table_hbm.at[pl.ds(0, n_rows)], rows_ref, sem).wait()


def _sc_gather(users, items, user_factors, item_factors):
  mesh = plsc.VectorSubcoreMesh(core_axis_name="c", subcore_axis_name="s")

  @functools.partial(
      pl.kernel,
      out_type=[
          jax.ShapeDtypeStruct((B_U, F), jnp.float32),
          jax.ShapeDtypeStruct((B_I, F), jnp.float32),
      ],
      mesh=mesh,
      scratch_types=[
          pltpu.VMEM((_U_PER_W,), jnp.int32),
          pltpu.VMEM((_I_PER_W,), jnp.int32),
          pltpu.VMEM((_U_PER_W, F), jnp.float32),
          pltpu.VMEM((_I_PER_W, F), jnp.float32),
          pltpu.SemaphoreType.DMA,
      ],
      compiler_params=pltpu.CompilerParams(needs_layout_passes=False),
  )
  def k(users_hbm, items_hbm, uf_hbm, if_hbm, u_out, v_out,
        idx_u, idx_i, rows_u, rows_i, sem):
    wid = lax.axis_index("s") * _NC + lax.axis_index("c")
    base_u = wid * _U_PER_W
    base_i = wid * _I_PER_W

    pltpu.sync_copy(users_hbm.at[pl.ds(base_u, _U_PER_W)], idx_u)
    pltpu.sync_copy(items_hbm.at[pl.ds(base_i, _I_PER_W)], idx_i)

    _gather_rows(uf_hbm, idx_u, rows_u, sem, _U_PER_W)
    _gather_rows(if_hbm, idx_i, rows_i, sem, _I_PER_W)
    _drain(uf_hbm, rows_u, sem, _U_PER_W)
    _drain(if_hbm, rows_i, sem, _I_PER_W)

    pltpu.sync_copy(rows_u, u_out.at[pl.ds(base_u, _U_PER_W)])
    pltpu.sync_copy(rows_i, v_out.at[pl.ds(base_i, _I_PER_W)])

  return k(users, items, user_factors, item_factors)


_BU_BLK = 1024


def _mm_body(u_ref, v_ref, o_ref):
  o_ref[...] = lax.dot_general(
      u_ref[...], v_ref[...],
      dimension_numbers=(((1,), (1,)), ((), ())),
      preferred_element_type=jnp.float32)


def _tc_matmul(u, v):
  return pl.pallas_call(
      _mm_body,
      grid=(B_U // _BU_BLK,),
      in_specs=[
          pl.BlockSpec((_BU_BLK, F), lambda i: (i, 0)),
          pl.BlockSpec((B_I, F), lambda i: (0, 0)),
      ],
      out_specs=pl.BlockSpec((_BU_BLK, B_I), lambda i: (i, 0)),
      out_shape=jax.ShapeDtypeStruct((B_U, B_I), jnp.float32),
      compiler_params=pltpu.CompilerParams(
          dimension_semantics=("arbitrary",)),
  )(u, v)


def kernel(users, items, user_factors, item_factors):
  users = users.astype(jnp.int32)
  items = items.astype(jnp.int32)
  u, v = _sc_gather(users, items, user_factors, item_factors)
  return _tc_matmul(u, v)
